# Initial kernel scaffold; baseline (speedup 1.0000x reference)
import jax
import jax.numpy as jnp
from jax.experimental import pallas as pl


def kernel(h, edge_index, edge_weight, W_in, W_layers, W_out):
    raise NotImplementedError("write your pallas kernel here")


# speedup vs baseline: 3.1368x; 3.1368x over previous
import functools

import jax
import jax.numpy as jnp
from jax import lax
from jax.experimental import pallas as pl
from jax.experimental.pallas import tpu as pltpu
from jax.experimental.pallas import tpu_sc as plsc

NC = 2
NS = 16
LANES = 16
K = 128
DH_HALF = 128


@functools.lru_cache(maxsize=None)
def _make_diffusion(n, ep):
  n_per_tile = ep // NS
  n_chunks = n_per_tile // K
  rows_per_tile = (n // NS) // 8 * 8
  rows_rem = n - NS * rows_per_tile
  mesh = plsc.VectorSubcoreMesh(core_axis_name="c", subcore_axis_name="s")

  @functools.partial(
      pl.kernel,
      mesh=mesh,
      out_type=jax.ShapeDtypeStruct((2 * n, DH_HALF), jnp.float32),
      scratch_types=[
          pltpu.VMEM((K,), jnp.int32),
          pltpu.VMEM((K,), jnp.int32),
          pltpu.VMEM((K,), jnp.float32),
          pltpu.VMEM((K, DH_HALF), jnp.float32),
          pltpu.VMEM_SHARED((n, DH_HALF), jnp.float32),
          pltpu.SemaphoreType.DMA,
      ],
  )
  def diffuse(x_hbm, src_hbm, dst_hbm, w_hbm, z_hbm, out_hbm,
              sidx, didx, wv, rows, acc, sem):
    c = lax.axis_index("c")
    s = lax.axis_index("s")

    r0 = s * rows_per_tile
    pltpu.sync_copy(z_hbm.at[pl.ds(r0, rows_per_tile)],
                    acc.at[pl.ds(r0, rows_per_tile)])
    if rows_rem:
      @pl.when(s == NS - 1)
      def _zero_tail():
        rt = NS * rows_per_tile
        pltpu.sync_copy(z_hbm.at[pl.ds(rt, rows_rem)],
                        acc.at[pl.ds(rt, rows_rem)])
    plsc.subcore_barrier()

    base0 = s * n_per_tile
    row_off = c * n

    def chunk_body(t, carry):
      base = pl.multiple_of(base0 + t * K, K)
      pltpu.sync_copy(src_hbm.at[pl.ds(base, K)], sidx)
      pltpu.sync_copy(dst_hbm.at[pl.ds(base, K)], didx)
      pltpu.sync_copy(w_hbm.at[pl.ds(base, K)], wv)
      for j in range(K // LANES):
        sl = pl.ds(j * LANES, LANES)
        sidx[sl] = sidx[sl] + row_off
      pltpu.async_copy(x_hbm.at[sidx], rows, sem).wait()

      def edge_body(g, carry2):
        w16 = wv[pl.ds(g * LANES, LANES)]
        for l in range(LANES):
          i = g * LANES + l
          wb = w16[l]
          for j in range(DH_HALF // LANES):
            sl = pl.ds(j * LANES, LANES)
            rows[i, sl] = rows[i, sl] * wb
        return carry2

      lax.fori_loop(0, K // LANES, edge_body, 0)

      pltpu.sync_copy(rows, acc.at[didx], add=True)
      return carry

    lax.fori_loop(0, n_chunks, chunk_body, 0)
    plsc.subcore_barrier()

    pltpu.sync_copy(acc.at[pl.ds(r0, rows_per_tile)],
                    out_hbm.at[pl.ds(c * n + r0, rows_per_tile)])
    if rows_rem:
      @pl.when(s == NS - 1)
      def _out_tail():
        rt = NS * rows_per_tile
        pltpu.sync_copy(acc.at[pl.ds(rt, rows_rem)],
                        out_hbm.at[pl.ds(c * n + rt, rows_rem)])

  return diffuse


def _mm_in(h, w_in):
  n, d_in = h.shape
  bn = 1000

  def body(h_ref, w_ref, o_ref):
    y = jnp.dot(h_ref[...], w_ref[...], preferred_element_type=jnp.float32)
    o_ref[0] = y[:, :DH_HALF]
    o_ref[1] = y[:, DH_HALF:]

  return pl.pallas_call(
      body,
      grid=(n // bn,),
      in_specs=[
          pl.BlockSpec((bn, d_in), lambda i: (i, 0)),
          pl.BlockSpec((d_in, 2 * DH_HALF), lambda i: (0, 0)),
      ],
      out_specs=pl.BlockSpec((2, bn, DH_HALF), lambda i: (0, i, 0)),
      out_shape=jax.ShapeDtypeStruct((2, n, DH_HALF), jnp.float32),
  )(h, w_in)


def _mm_layer(agg2, x2, w):
  n = x2.shape[1]
  bn = 1000

  def body(a_ref, x_ref, w_ref, o_ref):
    a = jnp.concatenate([a_ref[0], a_ref[1]], axis=1)
    x = jnp.concatenate([x_ref[0], x_ref[1]], axis=1)
    y = jnp.dot(a, w_ref[...], preferred_element_type=jnp.float32)
    y = y * jax.nn.sigmoid(y) + x
    o_ref[0] = y[:, :DH_HALF]
    o_ref[1] = y[:, DH_HALF:]

  spec2 = pl.BlockSpec((2, bn, DH_HALF), lambda i: (0, i, 0))
  return pl.pallas_call(
      body,
      grid=(n // bn,),
      in_specs=[spec2, spec2,
                pl.BlockSpec((2 * DH_HALF, 2 * DH_HALF), lambda i: (0, 0))],
      out_specs=spec2,
      out_shape=jax.ShapeDtypeStruct((2, n, DH_HALF), jnp.float32),
  )(agg2, x2, w)


def _mm_out(x2, w_out):
  n = x2.shape[1]
  d_out = w_out.shape[1]
  bn = 1000

  def body(x_ref, w_ref, o_ref):
    x = jnp.concatenate([x_ref[0], x_ref[1]], axis=1)
    o_ref[...] = jnp.dot(x, w_ref[...], preferred_element_type=jnp.float32)

  return pl.pallas_call(
      body,
      grid=(n // bn,),
      in_specs=[
          pl.BlockSpec((2, bn, DH_HALF), lambda i: (0, i, 0)),
          pl.BlockSpec((2 * DH_HALF, d_out), lambda i: (0, 0)),
      ],
      out_specs=pl.BlockSpec((bn, d_out), lambda i: (i, 0)),
      out_shape=jax.ShapeDtypeStruct((n, d_out), jnp.float32),
  )(x2, w_out)


def kernel(h, edge_index, edge_weight, W_in, W_layers, W_out):
  n = h.shape[0]
  e = edge_weight.shape[0]
  depth = W_layers.shape[0]

  ep = ((e + NS * K - 1) // (NS * K)) * (NS * K)
  src = edge_index[0]
  dst = edge_index[1]
  w = edge_weight
  if ep != e:
    pad = ep - e
    src = jnp.concatenate([src, jnp.zeros((pad,), jnp.int32)])
    dst = jnp.concatenate([dst, jnp.zeros((pad,), jnp.int32)])
    w = jnp.concatenate([w, jnp.zeros((pad,), jnp.float32)])

  zeros = jnp.zeros((n, DH_HALF), jnp.float32)
  diffuse = _make_diffusion(n, ep)

  x2 = _mm_in(h, W_in)
  for l in range(depth):
    agg = diffuse(x2.reshape(2 * n, DH_HALF), src, dst, w, zeros)
    x2 = _mm_layer(agg.reshape(2, n, DH_HALF), x2, W_layers[l])
  return _mm_out(x2, W_out)


# speedup vs baseline: 3.5207x; 1.1224x over previous
import functools

import jax
import jax.numpy as jnp
from jax import lax
from jax.experimental import pallas as pl
from jax.experimental.pallas import tpu as pltpu
from jax.experimental.pallas import tpu_sc as plsc

NC = 2
NS = 16
LANES = 16
K = 64
G = 16
DH_HALF = 128


@functools.lru_cache(maxsize=None)
def _make_diffusion(n, ep):
  n_per_tile = ep // NS
  n_chunks = n_per_tile // K
  n_groups = n_chunks // G
  assert n_chunks % 2 == 0 and n_chunks % G == 0 and n_groups >= 2
  rows_per_tile = (n // NS) // 8 * 8
  rows_rem = n - NS * rows_per_tile
  mesh = plsc.VectorSubcoreMesh(core_axis_name="c", subcore_axis_name="s")

  @functools.partial(
      pl.kernel,
      mesh=mesh,
      out_type=jax.ShapeDtypeStruct((2 * n, DH_HALF), jnp.float32),
      scratch_types=[
          pltpu.VMEM((2, G, K), jnp.int32),
          pltpu.VMEM((2, G, K), jnp.int32),
          pltpu.VMEM((2, G, K), jnp.float32),
          pltpu.VMEM((K, DH_HALF), jnp.float32),
          pltpu.VMEM((K, DH_HALF), jnp.float32),
          pltpu.VMEM((K, DH_HALF), jnp.float32),
          pltpu.VMEM((K, DH_HALF), jnp.float32),
          pltpu.VMEM_SHARED((n, DH_HALF), jnp.float32),
          pltpu.SemaphoreType.DMA,
          pltpu.SemaphoreType.DMA,
          pltpu.SemaphoreType.DMA,
          pltpu.SemaphoreType.DMA,
          pltpu.SemaphoreType.DMA,
      ],
  )
  def diffuse(x_hbm, src4, dst3, w3, z_hbm, out_hbm,
              sidx, didx, wts, rg0, rg1, rs0, rs1, acc,
              semg0, semg1, sems0, sems1, semt):
    c = lax.axis_index("c")
    s = lax.axis_index("s")
    rows_g = (rg0, rg1)
    rows_s = (rs0, rs1)
    semg = (semg0, semg1)
    sems = (sems0, sems1)

    def issue_tables(grp, slot):
      sl = pl.ds(grp * G, G)
      pltpu.async_copy(src4.at[c, s, sl], sidx.at[slot], semt)
      pltpu.async_copy(dst3.at[s, sl], didx.at[slot], semt)
      pltpu.async_copy(w3.at[s, sl], wts.at[slot], semt)

    def wait_tables(grp, slot):
      sl = pl.ds(grp * G, G)
      pltpu.make_async_copy(src4.at[c, s, sl], sidx.at[slot], semt).wait()
      pltpu.make_async_copy(dst3.at[s, sl], didx.at[slot], semt).wait()
      pltpu.make_async_copy(w3.at[s, sl], wts.at[slot], semt).wait()

    r0 = s * rows_per_tile
    pltpu.sync_copy(z_hbm.at[pl.ds(r0, rows_per_tile)],
                    acc.at[pl.ds(r0, rows_per_tile)])
    if rows_rem:
      @pl.when(s == NS - 1)
      def _zero_tail():
        rt = NS * rows_per_tile
        pltpu.sync_copy(z_hbm.at[pl.ds(rt, rows_rem)],
                        acc.at[pl.ds(rt, rows_rem)])

    issue_tables(0, 0)
    wait_tables(0, 0)
    plsc.subcore_barrier()

    for b in range(2):
      pltpu.async_copy(x_hbm.at[sidx.at[0, b]], rows_g[b], semg[b])

    def pair_body(p, carry):
      for b in range(2):
        t = p * 2 + b
        slot = (t // G) % 2
        tt = t % G
        pltpu.make_async_copy(x_hbm.at[sidx.at[slot, tt]], rows_g[b],
                              semg[b]).wait()
        @pl.when(p > 0)
        def _drain():
          pltpu.make_async_copy(rows_s[b], acc.at[didx.at[slot, tt]],
                                sems[b]).wait()

        def edge_body(g16, c2):
          w16 = wts[slot, tt, pl.ds(g16 * LANES, LANES)]
          for l in range(LANES):
            i = g16 * LANES + l
            wb = w16[l]
            for j in range(DH_HALF // LANES):
              fsl = pl.ds(j * LANES, LANES)
              rows_s[b][i, fsl] = rows_g[b][i, fsl] * wb
          return c2

        lax.fori_loop(0, K // LANES, edge_body, 0)

        pltpu.async_copy(rows_s[b], acc.at[didx.at[slot, tt]], sems[b],
                         add=True)

        @pl.when((tt == 1) & (t < (n_groups - 1) * G))
        def _refill():
          issue_tables(t // G + 1, 1 - slot)

        t2 = t + 2
        @pl.when((t2 % G == 0) & (t2 < n_chunks))
        def _tables_ready():
          wait_tables(t2 // G, (t2 // G) % 2)

        @pl.when(t2 < n_chunks)
        def _prefetch():
          pltpu.async_copy(x_hbm.at[sidx.at[(t2 // G) % 2, t2 % G]],
                           rows_g[b], semg[b])
      return carry

    lax.fori_loop(0, n_chunks // 2, pair_body, 0)
    for b in range(2):
      t = n_chunks - 2 + b
      slot = (t // G) % 2
      pltpu.make_async_copy(rows_s[b], acc.at[didx.at[slot, t % G]],
                            sems[b]).wait()
    plsc.subcore_barrier()

    pltpu.sync_copy(acc.at[pl.ds(r0, rows_per_tile)],
                    out_hbm.at[pl.ds(c * n + r0, rows_per_tile)])
    if rows_rem:
      @pl.when(s == NS - 1)
      def _out_tail():
        rt = NS * rows_per_tile
        pltpu.sync_copy(acc.at[pl.ds(rt, rows_rem)],
                        out_hbm.at[pl.ds(c * n + rt, rows_rem)])

  return diffuse


def _mm_in(h, w_in):
  n, d_in = h.shape
  bn = 1000

  def body(h_ref, w_ref, o_ref):
    y = jnp.dot(h_ref[...], w_ref[...], preferred_element_type=jnp.float32)
    o_ref[0] = y[:, :DH_HALF]
    o_ref[1] = y[:, DH_HALF:]

  return pl.pallas_call(
      body,
      grid=(n // bn,),
      in_specs=[
          pl.BlockSpec((bn, d_in), lambda i: (i, 0)),
          pl.BlockSpec((d_in, 2 * DH_HALF), lambda i: (0, 0)),
      ],
      out_specs=pl.BlockSpec((2, bn, DH_HALF), lambda i: (0, i, 0)),
      out_shape=jax.ShapeDtypeStruct((2, n, DH_HALF), jnp.float32),
  )(h, w_in)


def _mm_layer(agg2, x2, w):
  n = x2.shape[1]
  bn = 1000

  def body(a_ref, x_ref, w_ref, o_ref):
    a = jnp.concatenate([a_ref[0], a_ref[1]], axis=1)
    x = jnp.concatenate([x_ref[0], x_ref[1]], axis=1)
    y = jnp.dot(a, w_ref[...], preferred_element_type=jnp.float32)
    y = y * jax.nn.sigmoid(y) + x
    o_ref[0] = y[:, :DH_HALF]
    o_ref[1] = y[:, DH_HALF:]

  spec2 = pl.BlockSpec((2, bn, DH_HALF), lambda i: (0, i, 0))
  return pl.pallas_call(
      body,
      grid=(n // bn,),
      in_specs=[spec2, spec2,
                pl.BlockSpec((2 * DH_HALF, 2 * DH_HALF), lambda i: (0, 0))],
      out_specs=spec2,
      out_shape=jax.ShapeDtypeStruct((2, n, DH_HALF), jnp.float32),
  )(agg2, x2, w)


def _mm_out(x2, w_out):
  n = x2.shape[1]
  d_out = w_out.shape[1]
  bn = 1000

  def body(x_ref, w_ref, o_ref):
    x = jnp.concatenate([x_ref[0], x_ref[1]], axis=1)
    o_ref[...] = jnp.dot(x, w_ref[...], preferred_element_type=jnp.float32)

  return pl.pallas_call(
      body,
      grid=(n // bn,),
      in_specs=[
          pl.BlockSpec((2, bn, DH_HALF), lambda i: (0, i, 0)),
          pl.BlockSpec((2 * DH_HALF, d_out), lambda i: (0, 0)),
      ],
      out_specs=pl.BlockSpec((bn, d_out), lambda i: (i, 0)),
      out_shape=jax.ShapeDtypeStruct((n, d_out), jnp.float32),
  )(x2, w_out)


def kernel(h, edge_index, edge_weight, W_in, W_layers, W_out):
  n = h.shape[0]
  e = edge_weight.shape[0]
  depth = W_layers.shape[0]

  unit = NS * K * G
  ep = ((e + unit - 1) // unit) * unit
  src = edge_index[0]
  dst = edge_index[1]
  w = edge_weight
  if ep != e:
    pad = ep - e
    src = jnp.concatenate([src, jnp.zeros((pad,), jnp.int32)])
    dst = jnp.concatenate([dst, jnp.zeros((pad,), jnp.int32)])
    w = jnp.concatenate([w, jnp.zeros((pad,), jnp.float32)])

  n_chunks = ep // NS // K
  src3 = src.reshape(NS, n_chunks, K)
  src4 = jnp.stack([src3, src3 + n])
  dst3 = dst.reshape(NS, n_chunks, K)
  w3 = w.reshape(NS, n_chunks, K)
  zeros = jnp.zeros((n, DH_HALF), jnp.float32)
  diffuse = _make_diffusion(n, ep)

  x2 = _mm_in(h, W_in)
  for l in range(depth):
    agg = diffuse(x2.reshape(2 * n, DH_HALF), src4, dst3, w3, zeros)
    x2 = _mm_layer(agg.reshape(2, n, DH_HALF), x2, W_layers[l])
  return _mm_out(x2, W_out)


# speedup vs baseline: 3.6400x; 1.0339x over previous
import functools

import jax
import jax.numpy as jnp
from jax import lax
from jax.experimental import pallas as pl
from jax.experimental.pallas import tpu as pltpu
from jax.experimental.pallas import tpu_sc as plsc

NC = 2
NS = 16
LANES = 16
K = 32
G = 32
R = 4
DH_HALF = 128


@functools.lru_cache(maxsize=None)
def _make_diffusion(n, ep):
  n_per_tile = ep // NS
  n_chunks = n_per_tile // K
  n_groups = n_chunks // G
  assert n_chunks % R == 0 and n_chunks % G == 0 and n_groups >= 2
  assert G >= 2 * R
  rows_per_tile = (n // NS) // 8 * 8
  rows_rem = n - NS * rows_per_tile
  mesh = plsc.VectorSubcoreMesh(core_axis_name="c", subcore_axis_name="s")

  @functools.partial(
      pl.kernel,
      mesh=mesh,
      out_type=jax.ShapeDtypeStruct((2 * n, DH_HALF), jnp.float32),
      scratch_types=(
          [
              pltpu.VMEM((2, G * K // 128, 128), jnp.int32),
              pltpu.VMEM((2, G * K // 128, 128), jnp.int32),
              pltpu.VMEM((2, G * K // 128, 128), jnp.float32),
          ]
          + [pltpu.VMEM((K, DH_HALF), jnp.float32) for _ in range(2 * R)]
          + [pltpu.VMEM((K,), jnp.int32) for _ in range(R)]
          + [pltpu.VMEM_SHARED((n, DH_HALF), jnp.float32)]
          + [pltpu.SemaphoreType.DMA for _ in range(2 * R + 1)]
      ),
  )
  def diffuse(x_hbm, src4, dst3, w3, z_hbm, out_hbm,
              sidx, didx, wts, *rest):
    rows_g = rest[0:R]
    rows_s = rest[R:2 * R]
    dbuf = rest[2 * R:3 * R]
    acc = rest[3 * R]
    semg = rest[3 * R + 1:4 * R + 1]
    sems = rest[4 * R + 1:5 * R + 1]
    semt = rest[5 * R + 1]
    c = lax.axis_index("c")
    s = lax.axis_index("s")
    cpr = 128 // K

    grows = G * K // 128

    def issue_tables(grp, slot):
      sl = pl.ds(grp * grows, grows)
      pltpu.async_copy(src4.at[c, s, sl], sidx.at[slot], semt)
      pltpu.async_copy(dst3.at[s, sl], didx.at[slot], semt)
      pltpu.async_copy(w3.at[s, sl], wts.at[slot], semt)

    def wait_tables(grp, slot):
      sl = pl.ds(grp * grows, grows)
      pltpu.make_async_copy(src4.at[c, s, sl], sidx.at[slot], semt).wait()
      pltpu.make_async_copy(dst3.at[s, sl], didx.at[slot], semt).wait()
      pltpu.make_async_copy(w3.at[s, sl], wts.at[slot], semt).wait()

    r0 = s * rows_per_tile
    pltpu.sync_copy(z_hbm.at[pl.ds(r0, rows_per_tile)],
                    acc.at[pl.ds(r0, rows_per_tile)])
    if rows_rem:
      @pl.when(s == NS - 1)
      def _zero_tail():
        rt = NS * rows_per_tile
        pltpu.sync_copy(z_hbm.at[pl.ds(rt, rows_rem)],
                        acc.at[pl.ds(rt, rows_rem)])

    issue_tables(0, 0)
    wait_tables(0, 0)
    plsc.subcore_barrier()

    def gidx(t):
      slot = (t // G) % 2
      tt = t % G
      return sidx.at[slot, tt // cpr, pl.ds((tt % cpr) * K, K)]

    for b in range(R):
      pltpu.async_copy(x_hbm.at[gidx(b)], rows_g[b], semg[b])

    def ring_body(p, carry):
      for b in range(R):
        t = p * R + b
        slot = (t // G) % 2
        tt = t % G
        pltpu.make_async_copy(x_hbm.at[gidx(t)], rows_g[b], semg[b]).wait()
        @pl.when(p > 0)
        def _drain():
          pltpu.make_async_copy(rows_s[b], acc.at[dbuf[b]], sems[b]).wait()

        for j in range(K // LANES):
          dbuf[b][pl.ds(j * LANES, LANES)] = (
              didx[slot, tt // cpr, pl.ds((tt % cpr) * K + j * LANES, LANES)])

        def edge_body(g16, c2):
          w16 = wts[slot, tt // cpr,
                    pl.ds((tt % cpr) * K + g16 * LANES, LANES)]
          for l in range(LANES):
            i = g16 * LANES + l
            wb = w16[l]
            for j in range(DH_HALF // LANES):
              fsl = pl.ds(j * LANES, LANES)
              rows_s[b][i, fsl] = rows_g[b][i, fsl] * wb
          return c2

        lax.fori_loop(0, K // LANES, edge_body, 0)

        pltpu.async_copy(rows_s[b], acc.at[dbuf[b]], sems[b], add=True)

        @pl.when((tt == R - 1) & (t < (n_groups - 1) * G))
        def _refill():
          issue_tables(t // G + 1, 1 - slot)

        t2 = t + R
        @pl.when((t2 % G == 0) & (t2 < n_chunks))
        def _tables_ready():
          wait_tables(t2 // G, (t2 // G) % 2)

        @pl.when(t2 < n_chunks)
        def _prefetch():
          pltpu.async_copy(x_hbm.at[gidx(t2)], rows_g[b], semg[b])
      return carry

    lax.fori_loop(0, n_chunks // R, ring_body, 0)
    for b in range(R):
      pltpu.make_async_copy(rows_s[b], acc.at[dbuf[b]], sems[b]).wait()
    plsc.subcore_barrier()

    pltpu.sync_copy(acc.at[pl.ds(r0, rows_per_tile)],
                    out_hbm.at[pl.ds(c * n + r0, rows_per_tile)])
    if rows_rem:
      @pl.when(s == NS - 1)
      def _out_tail():
        rt = NS * rows_per_tile
        pltpu.sync_copy(acc.at[pl.ds(rt, rows_rem)],
                        out_hbm.at[pl.ds(c * n + rt, rows_rem)])

  return diffuse


def _mm_in(h, w_in):
  n, d_in = h.shape
  bn = 1000

  def body(h_ref, w_ref, o_ref):
    y = jnp.dot(h_ref[...], w_ref[...], preferred_element_type=jnp.float32)
    o_ref[0] = y[:, :DH_HALF]
    o_ref[1] = y[:, DH_HALF:]

  return pl.pallas_call(
      body,
      grid=(n // bn,),
      in_specs=[
          pl.BlockSpec((bn, d_in), lambda i: (i, 0)),
          pl.BlockSpec((d_in, 2 * DH_HALF), lambda i: (0, 0)),
      ],
      out_specs=pl.BlockSpec((2, bn, DH_HALF), lambda i: (0, i, 0)),
      out_shape=jax.ShapeDtypeStruct((2, n, DH_HALF), jnp.float32),
  )(h, w_in)


def _mm_layer(agg2, x2, w):
  n = x2.shape[1]
  bn = 1000

  def body(a_ref, x_ref, w_ref, o_ref):
    a = jnp.concatenate([a_ref[0], a_ref[1]], axis=1)
    x = jnp.concatenate([x_ref[0], x_ref[1]], axis=1)
    y = jnp.dot(a, w_ref[...], preferred_element_type=jnp.float32)
    y = y * jax.nn.sigmoid(y) + x
    o_ref[0] = y[:, :DH_HALF]
    o_ref[1] = y[:, DH_HALF:]

  spec2 = pl.BlockSpec((2, bn, DH_HALF), lambda i: (0, i, 0))
  return pl.pallas_call(
      body,
      grid=(n // bn,),
      in_specs=[spec2, spec2,
                pl.BlockSpec((2 * DH_HALF, 2 * DH_HALF), lambda i: (0, 0))],
      out_specs=spec2,
      out_shape=jax.ShapeDtypeStruct((2, n, DH_HALF), jnp.float32),
  )(agg2, x2, w)


def _mm_out(x2, w_out):
  n = x2.shape[1]
  d_out = w_out.shape[1]
  bn = 1000

  def body(x_ref, w_ref, o_ref):
    x = jnp.concatenate([x_ref[0], x_ref[1]], axis=1)
    o_ref[...] = jnp.dot(x, w_ref[...], preferred_element_type=jnp.float32)

  return pl.pallas_call(
      body,
      grid=(n // bn,),
      in_specs=[
          pl.BlockSpec((2, bn, DH_HALF), lambda i: (0, i, 0)),
          pl.BlockSpec((2 * DH_HALF, d_out), lambda i: (0, 0)),
      ],
      out_specs=pl.BlockSpec((bn, d_out), lambda i: (i, 0)),
      out_shape=jax.ShapeDtypeStruct((n, d_out), jnp.float32),
  )(x2, w_out)


def kernel(h, edge_index, edge_weight, W_in, W_layers, W_out):
  n = h.shape[0]
  e = edge_weight.shape[0]
  depth = W_layers.shape[0]

  unit = NS * K * G
  ep = ((e + unit - 1) // unit) * unit
  src = edge_index[0]
  dst = edge_index[1]
  w = edge_weight
  if ep != e:
    pad = ep - e
    src = jnp.concatenate([src, jnp.zeros((pad,), jnp.int32)])
    dst = jnp.concatenate([dst, jnp.zeros((pad,), jnp.int32)])
    w = jnp.concatenate([w, jnp.zeros((pad,), jnp.float32)])

  src3 = src.reshape(NS, -1, 128)
  src4 = jnp.stack([src3, src3 + n])
  dst3 = dst.reshape(NS, -1, 128)
  w3 = w.reshape(NS, -1, 128)
  zeros = jnp.zeros((n, DH_HALF), jnp.float32)
  diffuse = _make_diffusion(n, ep)

  x2 = _mm_in(h, W_in)
  for l in range(depth):
    agg = diffuse(x2.reshape(2 * n, DH_HALF), src4, dst3, w3, zeros)
    x2 = _mm_layer(agg.reshape(2, n, DH_HALF), x2, W_layers[l])
  return _mm_out(x2, W_out)
